# Initial kernel scaffold; baseline (speedup 1.0000x reference)
import functools

import jax
import jax.numpy as jnp
from jax import lax
from jax.experimental import pallas as pl
from jax.experimental.pallas import tpu as pltpu
from jax.experimental.pallas import tpu_sc as plsc

B = 2048
F = 32
L = 16
TBL = 2048
NCORN = F + 1
GROWTH = 1.3
_PRIMES = [2654436881, 5915587277, 1500450271, 3267000013, 5754853343,
           4093082899, 9576890767, 3628273133, 2860486313, 5463458053,
           3367900313, 5654500741, 5654500763, 5654500771, 5654500783,
           5654500801, 5654500811, 5654500861, 5654500879, 5654500889,
           5654500897, 5654500927, 5654500961, 5654500981, 5654500993,
           9999999967, 7654487179, 7654489553, 7654495087, 7654486423,
           7654488209, 8654487029]


def _encode_body(p_ref, pm_ref, idx_ref, w_ref):
    l = pl.program_id(0)
    p = p_ref[0]
    pm = pm_ref[...][:, 0:1]
    fl = jnp.floor(p)
    fr = p - fl
    iota_i = lax.broadcasted_iota(jnp.int32, (F, B), 0)
    m = jnp.zeros((F, B), jnp.int32)
    r = jnp.zeros((F, B), jnp.int32)
    for k in range(F):
        frk = fr[k:k + 1, :]
        m = m + (frk <= fr).astype(jnp.int32)
        r = r + (frk < fr).astype(jnp.int32) \
              + jnp.where((frk == fr) & (iota_i > k), 1, 0)
    cs = jnp.zeros((F, B), jnp.float32)
    for k in range(F):
        cs = cs + jnp.where(r[k:k + 1, :] == iota_i, fr[k:k + 1, :], 0.0)
    zf = jnp.zeros((1, B), jnp.float32)
    wfull = cs - jnp.concatenate([zf, cs[:F - 1, :]], axis=0)
    w_ref[0, pl.ds(0, F), :] = wfull
    w_ref[0, pl.ds(F, 1), :] = 1.0 - cs[F - 1:F, :]
    zi = jnp.zeros((1, B), jnp.int32)
    fli = fl.astype(jnp.int32)
    dfl = fli - jnp.concatenate([zi, fli[:F - 1, :]], axis=0)
    msh = jnp.concatenate([zi, m[:F - 1, :]], axis=0)
    tz = (dfl * pm) & 2047
    lofs = l * TBL
    for j in range(NCORN):
        dind = (m > j).astype(jnp.int32) - (msh > j).astype(jnp.int32)
        t = (tz + dind * pm) & 2047
        t = t[:16, :] ^ t[16:, :]
        t = t[:8, :] ^ t[8:, :]
        t = t[:4, :] ^ t[4:, :]
        t = t[:2, :] ^ t[2:, :]
        t = t[0:1, :] ^ t[1:2, :]
        idx_ref[0, pl.ds(j, 1), :] = t + lofs


_encode = pl.pallas_call(
    _encode_body,
    grid=(L,),
    in_specs=[
        pl.BlockSpec((1, F, B), lambda l: (l, 0, 0)),
        pl.BlockSpec((F, 128), lambda l: (0, 0)),
    ],
    out_specs=[
        pl.BlockSpec((1, NCORN, B), lambda l: (l, 0, 0)),
        pl.BlockSpec((1, NCORN, B), lambda l: (l, 0, 0)),
    ],
    out_shape=[
        jax.ShapeDtypeStruct((L, NCORN, B), jnp.int32),
        jax.ShapeDtypeStruct((L, NCORN, B), jnp.float32),
    ],
)

NW = 32
BPW = B // NW
_mesh = plsc.VectorSubcoreMesh(core_axis_name="c", subcore_axis_name="s")


@functools.partial(
    pl.kernel,
    out_type=jax.ShapeDtypeStruct((2 * L, B), jnp.float32),
    mesh=_mesh,
    scratch_types=[
        pltpu.VMEM((L * TBL,), jnp.float32),
        pltpu.VMEM((L * TBL,), jnp.float32),
        pltpu.VMEM((2, NCORN, BPW), jnp.int32),
        pltpu.VMEM((2, NCORN, BPW), jnp.float32),
        pltpu.VMEM((2 * L, BPW), jnp.float32),
        pltpu.SemaphoreType.DMA,
        pltpu.SemaphoreType.DMA,
        pltpu.SemaphoreType.DMA,
        pltpu.SemaphoreType.DMA,
    ],
)
def _sc_gather(t0_hbm, t1_hbm, idx_hbm, w_hbm, out_hbm,
               t0_v, t1_v, idx_v, w_v, f_v,
               sem_i0, sem_w0, sem_i1, sem_w1):
    cid = lax.axis_index("c")
    sid = lax.axis_index("s")
    base = (sid * 2 + cid) * BPW
    pltpu.sync_copy(t0_hbm, t0_v)
    pltpu.sync_copy(t1_hbm, t1_v)
    sems = ((sem_i0, sem_w0), (sem_i1, sem_w1))

    def fire(lv, slot):
        ci = pltpu.async_copy(idx_hbm.at[lv, :, pl.ds(base, BPW)],
                              idx_v.at[slot], sems[slot][0])
        cw = pltpu.async_copy(w_hbm.at[lv, :, pl.ds(base, BPW)],
                              w_v.at[slot], sems[slot][1])
        return ci, cw

    pend = fire(0, 0)
    for lv in range(L):
        slot = lv % 2
        ci, cw = pend
        ci.wait()
        cw.wait()
        if lv + 1 < L:
            pend = fire(lv + 1, (lv + 1) % 2)

        def jbody(j, accs, _slot=slot):
            out = []
            for s in range(4):
                iv = idx_v[_slot, j, pl.ds(s * 16, 16)]
                wv = w_v[_slot, j, pl.ds(s * 16, 16)]
                g0 = plsc.load_gather(t0_v, [iv])
                g1 = plsc.load_gather(t1_v, [iv])
                out.append(accs[2 * s] + wv * g0)
                out.append(accs[2 * s + 1] + wv * g1)
            return tuple(out)

        accs = tuple(jnp.zeros((16,), jnp.float32) for _ in range(8))
        accs = lax.fori_loop(0, NCORN, jbody, accs)
        for s in range(4):
            f_v[2 * lv, pl.ds(s * 16, 16)] = accs[2 * s]
            f_v[2 * lv + 1, pl.ds(s * 16, 16)] = accs[2 * s + 1]
    pltpu.sync_copy(f_v, out_hbm.at[:, pl.ds(base, BPW)])


BBLK = 512


def _mlp_body(x_ref, f_ref, w1x_ref, w1f_ref, b1_ref, w2_ref, b2_ref,
              wa_ref, wv_ref, act_ref, val_ref):
    dot = functools.partial(jnp.dot, preferred_element_type=jnp.float32,
                            precision=lax.Precision.HIGHEST)
    hp = dot(w1x_ref[...], x_ref[...]) + dot(w1f_ref[...], f_ref[...])
    h1 = jnp.maximum(hp + b1_ref[...], 0.0)
    h2 = jnp.maximum(dot(w2_ref[...], h1) + b2_ref[...], 0.0)
    act_ref[...] = dot(wa_ref[...], h2)
    val_ref[...] = dot(wv_ref[...], h2)


_mlp = pl.pallas_call(
    _mlp_body,
    grid=(B // BBLK,),
    in_specs=[
        pl.BlockSpec((F, BBLK), lambda n: (0, n)),
        pl.BlockSpec((2 * L, BBLK), lambda n: (0, n)),
        pl.BlockSpec((2048, F), lambda n: (0, 0)),
        pl.BlockSpec((2048, 2 * L), lambda n: (0, 0)),
        pl.BlockSpec((2048, 1), lambda n: (0, 0)),
        pl.BlockSpec((2048, 2048), lambda n: (0, 0)),
        pl.BlockSpec((2048, 1), lambda n: (0, 0)),
        pl.BlockSpec((4, 2048), lambda n: (0, 0)),
        pl.BlockSpec((1, 2048), lambda n: (0, 0)),
    ],
    out_specs=[
        pl.BlockSpec((4, BBLK), lambda n: (0, n)),
        pl.BlockSpec((1, BBLK), lambda n: (0, n)),
    ],
    out_shape=[
        jax.ShapeDtypeStruct((4, B), jnp.float32),
        jax.ShapeDtypeStruct((1, B), jnp.float32),
    ],
)


def kernel(x, T, W1, b1, W2, b2, Wa, ba, Wv, bv):
    K = jnp.power(jnp.float32(GROWTH), jnp.arange(L, dtype=jnp.float32))
    cum = jnp.cumsum(x, -1)
    p = cum[:, None, :] * K[:, None]
    pT = jnp.transpose(p, (1, 2, 0))
    pmarr = jnp.broadcast_to(
        jnp.array([q & 2047 for q in _PRIMES], jnp.int32)[:, None], (F, 128))
    idxT, wT = _encode(pT, pmarr)
    t0 = jnp.reshape(T[:, :, 0], (L * TBL,))
    t1 = jnp.reshape(T[:, :, 1], (L * TBL,))
    fraw = _sc_gather(t0, t1, idxT, wT)
    krow = jnp.repeat(K, 2)[:, None]
    fT = fraw / krow
    actT, valT = _mlp(x.T, fT, W1[:, :F], W1[:, F:], b1[:, None],
                      W2, b2[:, None], Wa, Wv)
    action = actT.T + ba
    value = valT.T + bv
    return (action, value)


# speedup vs baseline: 35.0816x; 35.0816x over previous
import functools

import jax
import jax.numpy as jnp
from jax import lax
from jax.experimental import pallas as pl
from jax.experimental.pallas import tpu as pltpu
from jax.experimental.pallas import tpu_sc as plsc

B = 2048
F = 32
L = 16
TBL = 2048
NCORN = F + 1
GROWTH = 1.3
def _i0(_=None):
    return jnp.int32(0)
_PRIMES = [2654436881, 5915587277, 1500450271, 3267000013, 5754853343,
           4093082899, 9576890767, 3628273133, 2860486313, 5463458053,
           3367900313, 5654500741, 5654500763, 5654500771, 5654500783,
           5654500801, 5654500811, 5654500861, 5654500879, 5654500889,
           5654500897, 5654500927, 5654500961, 5654500981, 5654500993,
           9999999967, 7654487179, 7654489553, 7654495087, 7654486423,
           7654488209, 8654487029]


BCHK = 512


def _encode_body(p_ref, pm_ref, idx_ref, w_ref):
    l = pl.program_id(0)
    p = p_ref[0]
    pm = pm_ref[...][:, 0:1]
    fl = jnp.floor(p)
    fr = p - fl
    iota32 = lax.broadcasted_iota(jnp.int32, (F, BCHK), 0)
    macc = [jnp.zeros((F, BCHK), jnp.int16) for _ in range(4)]
    racc = [jnp.zeros((F, BCHK), jnp.int16) for _ in range(4)]
    for k in range(F):
        a = k % 4
        frk = fr[k:k + 1, :]
        macc[a] = macc[a] + (frk <= fr).astype(jnp.int16)
        racc[a] = racc[a] + (frk < fr).astype(jnp.int16) \
            + ((frk == fr) & (iota32 > k)).astype(jnp.int16)
    m = (macc[0] + macc[1]) + (macc[2] + macc[3])
    r = (racc[0] + racc[1]) + (racc[2] + racc[3])
    r32 = r.astype(jnp.int32)
    csa = [jnp.zeros((F, BCHK), jnp.float32) for _ in range(4)]
    for k in range(F):
        csa[k % 4] = csa[k % 4] + jnp.where(r32[k:k + 1, :] == iota32,
                                            fr[k:k + 1, :], 0.0)
    cs = (csa[0] + csa[1]) + (csa[2] + csa[3])
    zf = jnp.zeros((1, BCHK), jnp.float32)
    wfull = cs - jnp.concatenate([zf, cs[:F - 1, :]], axis=0)
    w_ref[0, pl.ds(0, F), :] = wfull
    w_ref[0, pl.ds(F, 1), :] = 1.0 - cs[F - 1:F, :]
    zi16 = jnp.zeros((1, BCHK), jnp.int16)
    fli = fl.astype(jnp.int32).astype(jnp.int16)
    dfl = fli - jnp.concatenate([zi16, fli[:F - 1, :]], axis=0)
    msh = jnp.concatenate([zi16, m[:F - 1, :]], axis=0)
    pm16 = pm.astype(jnp.int16)
    tz = (dfl * pm16) & 2047
    cp = tz ^ ((tz + pm16) & 2047)
    cm = tz ^ ((tz - pm16) & 2047)
    z16 = jnp.zeros((F, BCHK), jnp.int16)
    lofs = l * TBL
    for j in range(NCORN):
        ind = m > j
        inds = msh > j
        t = tz ^ jnp.where(ind & (~inds), cp, z16) \
               ^ jnp.where(inds & (~ind), cm, z16)
        t = t[:16, :] ^ t[16:, :]
        t = t[:8, :] ^ t[8:, :]
        t = t[:4, :] ^ t[4:, :]
        t = t[:2, :] ^ t[2:, :]
        t = t[0:1, :] ^ t[1:2, :]
        idx_ref[0, pl.ds(j, 1), :] = t.astype(jnp.int32) + lofs


_encode = pl.pallas_call(
    _encode_body,
    grid=(L, B // BCHK),
    in_specs=[
        pl.BlockSpec((1, F, BCHK), lambda l, n: (l, _i0(), n)),
        pl.BlockSpec((F, 128), lambda l, n: (_i0(), _i0())),
    ],
    out_specs=[
        pl.BlockSpec((1, NCORN, BCHK), lambda l, n: (l, _i0(), n)),
        pl.BlockSpec((1, NCORN, BCHK), lambda l, n: (l, _i0(), n)),
    ],
    out_shape=[
        jax.ShapeDtypeStruct((L, NCORN, B), jnp.int32),
        jax.ShapeDtypeStruct((L, NCORN, B), jnp.float32),
    ],
)

LH = L // 2
BPW = 128
CHK = NCORN * BPW
_mesh = plsc.VectorSubcoreMesh(core_axis_name="c", subcore_axis_name="s")


@functools.partial(
    pl.kernel,
    out_type=jax.ShapeDtypeStruct((L * 16 * 2 * BPW,), jnp.float32),
    mesh=_mesh,
    scratch_types=[
        pltpu.VMEM((LH * TBL,), jnp.float32),
        pltpu.VMEM((LH * TBL,), jnp.float32),
        pltpu.VMEM((2 * CHK,), jnp.int32),
        pltpu.VMEM((2 * CHK,), jnp.float32),
        pltpu.VMEM((LH * 2 * BPW,), jnp.float32),
        pltpu.SemaphoreType.DMA,
        pltpu.SemaphoreType.DMA,
        pltpu.SemaphoreType.DMA,
        pltpu.SemaphoreType.DMA,
        pltpu.SemaphoreType.DMA,
    ],
    compiler_params=pltpu.CompilerParams(needs_layout_passes=False),
)
def _sc_gather(t0_hbm, t1_hbm, idx_hbm, w_hbm, out_hbm,
               t0_v, t1_v, idx_v, w_v, f_v,
               sem_i0, sem_w0, sem_i1, sem_w1, sem_o):
    cid = lax.axis_index("c")
    sid = lax.axis_index("s")
    tbase = cid * jnp.int32(LH * TBL)
    pltpu.sync_copy(t0_hbm.at[pl.ds(tbase, LH * TBL)], t0_v)
    pltpu.sync_copy(t1_hbm.at[pl.ds(tbase, LH * TBL)], t1_v)
    sems = ((sem_i0, sem_w0), (sem_i1, sem_w1))
    lanes = lax.iota(jnp.int32, 16)

    def fire(lv, slot):
        lvl = cid * LH + jnp.int32(lv)
        off = (lvl * 16 + sid) * CHK
        ci = pltpu.async_copy(idx_hbm.at[pl.ds(off, CHK)],
                              idx_v.at[pl.ds(jnp.int32(slot * CHK), CHK)],
                              sems[slot][0])
        cw = pltpu.async_copy(w_hbm.at[pl.ds(off, CHK)],
                              w_v.at[pl.ds(jnp.int32(slot * CHK), CHK)],
                              sems[slot][1])
        return ci, cw

    pend = fire(0, 0)
    out_copies = []
    for lv in range(LH):
        slot = lv % 2
        ci, cw = pend
        ci.wait()
        cw.wait()
        if lv + 1 < LH:
            pend = fire(lv + 1, (lv + 1) % 2)

        def jbody(j, accs, _slot=slot):
            jo = jnp.int32(_slot * CHK) + j * BPW
            out = []
            for s in range(8):
                addr = jo + s * 16 + lanes
                iv = plsc.load_gather(idx_v, [addr]) - tbase
                wv = plsc.load_gather(w_v, [addr])
                g0 = plsc.load_gather(t0_v, [iv])
                g1 = plsc.load_gather(t1_v, [iv])
                out.append(accs[2 * s] + wv * g0)
                out.append(accs[2 * s + 1] + wv * g1)
            return tuple(out)

        accs = tuple(jnp.zeros((16,), jnp.float32) for _ in range(16))
        accs = lax.fori_loop(jnp.int32(0), jnp.int32(NCORN), jbody, accs)
        for s in range(8):
            f_v[pl.ds(lv * 2 * BPW + s * 16, 16)] = accs[2 * s]
            f_v[pl.ds(lv * 2 * BPW + BPW + s * 16, 16)] = accs[2 * s + 1]
        lvl = cid * LH + jnp.int32(lv)
        oofs = (lvl * 16 + sid) * jnp.int32(2 * BPW)
        out_copies.append(
            pltpu.async_copy(f_v.at[pl.ds(jnp.int32(lv * 2 * BPW), 2 * BPW)],
                             out_hbm.at[pl.ds(oofs, 2 * BPW)], sem_o))
    for c in out_copies:
        c.wait()


BBLK = 1024


def _mlp_body(x_ref, f_ref, w1x_ref, w1f_ref, b1_ref, w2_ref, b2_ref,
              wav_ref, av_ref):
    dot = functools.partial(jnp.dot, preferred_element_type=jnp.float32)
    bf = jnp.bfloat16
    hp = dot(w1x_ref[...], x_ref[...].astype(bf)) \
        + dot(w1f_ref[...], f_ref[...].astype(bf))
    h1 = jnp.maximum(hp + b1_ref[...], 0.0).astype(bf)
    h2 = jnp.maximum(dot(w2_ref[...], h1) + b2_ref[...], 0.0).astype(bf)
    av_ref[...] = dot(wav_ref[...], h2)


_mlp = pl.pallas_call(
    _mlp_body,
    grid=(B // BBLK,),
    in_specs=[
        pl.BlockSpec((F, BBLK), lambda n: (_i0(), n)),
        pl.BlockSpec((2 * L, BBLK), lambda n: (_i0(), n)),
        pl.BlockSpec((2048, F), lambda n: (_i0(), _i0())),
        pl.BlockSpec((2048, 2 * L), lambda n: (_i0(), _i0())),
        pl.BlockSpec((2048, 1), lambda n: (_i0(), _i0())),
        pl.BlockSpec((2048, 2048), lambda n: (_i0(), _i0())),
        pl.BlockSpec((2048, 1), lambda n: (_i0(), _i0())),
        pl.BlockSpec((5, 2048), lambda n: (_i0(), _i0())),
    ],
    out_specs=pl.BlockSpec((5, BBLK), lambda n: (_i0(), n)),
    out_shape=jax.ShapeDtypeStruct((5, B), jnp.float32),
)


def kernel(x, T, W1, b1, W2, b2, Wa, ba, Wv, bv):
    K = jnp.power(jnp.float32(GROWTH), jnp.arange(L, dtype=jnp.float32))
    cum = jnp.cumsum(x, -1)
    p = cum[:, None, :] * K[:, None]
    pT = jnp.transpose(p, (1, 2, 0))
    pmarr = jnp.broadcast_to(
        jnp.array([q & 2047 for q in _PRIMES], jnp.int32)[:, None], (F, 128))
    idxT, wT = _encode(pT, pmarr)
    t0 = jnp.reshape(T[:, :, 0], (L * TBL,))
    t1 = jnp.reshape(T[:, :, 1], (L * TBL,))
    idx_f = jnp.transpose(idxT.reshape(L, NCORN, 16, BPW),
                          (0, 2, 1, 3)).reshape(-1)
    w_f = jnp.transpose(wT.reshape(L, NCORN, 16, BPW),
                        (0, 2, 1, 3)).reshape(-1)
    fraw = _sc_gather(t0, t1, idx_f, w_f)
    krow = jnp.repeat(K, 2)[:, None]
    fT = jnp.transpose(fraw.reshape(L, 16, 2, BPW),
                       (0, 2, 1, 3)).reshape(2 * L, B) / krow
    bf = jnp.bfloat16
    wav = jnp.concatenate([Wa, Wv], axis=0).astype(bf)
    avT = _mlp(x.T, fT, W1[:, :F].astype(bf), W1[:, F:].astype(bf),
               b1[:, None], W2.astype(bf), b2[:, None], wav)
    action = avT[:4].T + ba
    value = avT[4:5].T + bv
    return (action, value)
